# Initial kernel scaffold; baseline (speedup 1.0000x reference)
import jax
import jax.numpy as jnp
from jax.experimental import pallas as pl


def kernel(token_ids, weight):
    raise NotImplementedError("write your pallas kernel here")


# speedup vs baseline: 1.4979x; 1.4979x over previous
import jax
import jax.numpy as jnp
from jax import lax
from jax.experimental import pallas as pl
from jax.experimental.pallas import tpu as pltpu
from jax.experimental.pallas import tpu_sc as plsc

NC = 2
NS = 16
NW = NC * NS
GA = 128
G = 10
NBUF = 2


def _body(idx_hbm, table_hbm, out_hbm, idx_v, rows_v, sem_g, sem_s0, sem_s1):
    ng = idx_hbm.shape[1]
    wid = lax.axis_index("s") * NC + lax.axis_index("c")
    pltpu.sync_copy(idx_hbm.at[wid], idx_v)
    sem_s = (sem_s0, sem_s1)

    @pl.loop(0, ng // NBUF)
    def _outer(i):
        for b in range(NBUF):
            g = i * NBUF + b

            @pl.when(g >= NBUF)
            def _():
                pltpu.make_async_copy(
                    rows_v.at[b], out_hbm.at[wid, g - NBUF], sem_s[b]
                ).wait()

            for j in range(G):
                pltpu.async_copy(
                    table_hbm.at[idx_v.at[g, j]], rows_v.at[b, j], sem_g
                )
            pltpu.make_async_copy(out_hbm.at[wid, g], rows_v.at[b], sem_g).wait()
            pltpu.async_copy(rows_v.at[b], out_hbm.at[wid, g], sem_s[b])

    for b in range(NBUF):
        g = ng - NBUF + b
        pltpu.make_async_copy(rows_v.at[b], out_hbm.at[wid, g], sem_s[b]).wait()


def kernel(token_ids, weight):
    d = weight.shape[1]
    total = 1
    for s in token_ids.shape:
        total *= s
    per_w = total // NW
    ng = per_w // (G * GA)
    assert total == NW * ng * G * GA and ng % NBUF == 0

    ids = token_ids.reshape(-1).astype(jnp.int32).reshape(NW, ng, G, GA)

    k = pl.kernel(
        _body,
        out_type=jax.ShapeDtypeStruct((NW, ng, G, GA, d), jnp.float32),
        mesh=plsc.VectorSubcoreMesh(core_axis_name="c", subcore_axis_name="s"),
        compiler_params=pltpu.CompilerParams(use_tc_tiling_on_sc=False),
        scratch_types=[
            pltpu.VMEM((ng, G, GA), jnp.int32),
            pltpu.VMEM((NBUF, G, GA, d), jnp.float32),
            pltpu.SemaphoreType.DMA,
            pltpu.SemaphoreType.DMA,
            pltpu.SemaphoreType.DMA,
        ],
    )
    out = k(ids, weight)
    return out.reshape(*token_ids.shape, d)
